# baseline (speedup 1.0000x reference)
import functools

import jax
import jax.numpy as jnp
from jax import lax
from jax.experimental import pallas as pl
from jax.experimental.pallas import tpu as pltpu
from jax.experimental.pallas import tpu_sc as plsc

_BB = 128
_CH = 128
_NCHUNK = 2


def _attn_block_kernel(nm, e_ref, wg_ref, zqg_ref, rw_ref, w1t_ref, w2t_ref,
                       b_ref, u_ref, out_ref):
    bb, m = nm
    e = e_ref[...]
    nh = _unpack_rows(wg_ref[...])
    dim = e.shape[-1]
    ones_c = jnp.ones((dim, 1), jnp.float32)
    dew = jnp.dot(e * nh, ones_c, preferred_element_type=jnp.float32)
    etr = e - dew * nh
    x = jnp.dot(etr.astype(jnp.bfloat16), w2t_ref[...],
                preferred_element_type=jnp.float32)
    zw = jnp.dot(zqg_ref[...], w1t_ref[...],
                 preferred_element_type=jnp.float32) + b_ref[...]
    x2 = x + jnp.broadcast_to(zw[:, None, :], (bb, m, zw.shape[-1])
                              ).reshape(bb * m, zw.shape[-1])
    t2 = jnp.tanh(x2.astype(jnp.bfloat16))
    logits = jnp.dot(t2, u_ref[...], preferred_element_type=jnp.float32)
    l3 = logits.reshape(bb, m, 1)
    mx = jnp.max(l3, axis=1, keepdims=True)
    ex = jnp.exp(l3 - mx)
    alpha = ex / jnp.sum(ex, axis=1, keepdims=True)
    attn = alpha + rw_ref[...].reshape(bb, m, 1)
    etr3 = etr.reshape(bb, m, dim)
    out_ref[...] = jnp.sum(attn * etr3, axis=1)


def _normalize_kernel(w_ref, out_ref):
    w = w_ref[...]
    ssq = jnp.sum(w * w, axis=-1, keepdims=True)
    out_ref[...] = w / jnp.maximum(jnp.sqrt(ssq), 1e-12)


def _normalize_call(w):
    return pl.pallas_call(
        _normalize_kernel,
        out_shape=jax.ShapeDtypeStruct(w.shape, jnp.float32),
    )(w)


def _pack_table(nhat):
    h = nhat.shape[-1] // 2
    stored = jnp.stack([nhat[:, :h], nhat[:, h:]], axis=-1).astype(jnp.bfloat16)
    return jax.lax.bitcast_convert_type(stored, jnp.int32)


def _unpack_rows(u):
    f_lo = jax.lax.bitcast_convert_type(jnp.left_shift(u, 16), jnp.float32)
    f_hi = jax.lax.bitcast_convert_type(u & jnp.int32(-65536), jnp.float32)
    return jnp.concatenate([f_lo, f_hi], axis=-1)


@functools.cache
def _make_sc_gather(n_rows, n_q, dim):
    info = plsc.get_sparse_core_info()
    nc = info.num_cores
    nw = nc * info.num_subcores
    per_w = n_rows // nw
    n_ch = per_w // _CH
    per_wq = n_q // nw
    mesh = plsc.VectorSubcoreMesh(core_axis_name="c", subcore_axis_name="s")

    @functools.partial(
        pl.kernel, mesh=mesh,
        out_type=[jax.ShapeDtypeStruct((n_rows, dim // 2), jnp.int32),
                  jax.ShapeDtypeStruct((n_q, dim), jnp.float32)],
        scratch_types=[pltpu.VMEM((per_w,), jnp.int32),
                       pltpu.VMEM((_CH, dim // 2), jnp.int32),
                       pltpu.VMEM((_CH, dim // 2), jnp.int32),
                       pltpu.VMEM((per_wq,), jnp.int32),
                       pltpu.VMEM((per_wq, dim), jnp.float32),
                       pltpu.SemaphoreType.DMA,
                       pltpu.SemaphoreType.DMA,
                       pltpu.SemaphoreType.DMA,
                       pltpu.SemaphoreType.DMA,
                       pltpu.SemaphoreType.DMA],
    )
    def gk(rid_hbm, qrid_hbm, wr_hbm, zq_hbm, outw, outz,
           idx_all, rows0, rows1, qidx_v, qrows_v, g0, g1, s0, s1, qsem):
        wid = lax.axis_index("s") * nc + lax.axis_index("c")
        base0 = wid * per_w
        pltpu.sync_copy(rid_hbm.at[pl.ds(pl.multiple_of(base0, 8), per_w)],
                        idx_all)
        bufs = (rows0, rows1)
        gsems = (g0, g1)
        ssems = (s0, s1)

        def gather_start(i, buf, sem):
            return pltpu.async_copy(
                wr_hbm.at[idx_all.at[pl.ds(i * _CH, _CH)]], buf, sem)

        def store_start(i, buf, sem):
            dst = pl.multiple_of(base0 + i * _CH, 8)
            return pltpu.async_copy(buf, outw.at[pl.ds(dst, _CH)], sem)

        pend_g = gather_start(0, bufs[0], gsems[0])
        stores = [None, None]
        for i in range(n_ch):
            cur = i & 1
            nxt = cur ^ 1
            pend_g.wait()
            st = store_start(i, bufs[cur], ssems[cur])
            if i + 1 < n_ch:
                if stores[nxt] is not None:
                    stores[nxt].wait()
                pend_g = gather_start(i + 1, bufs[nxt], gsems[nxt])
            stores[cur] = st
        stores[0].wait()
        stores[1].wait()

        qbase = pl.multiple_of(wid * per_wq, 8)
        pltpu.sync_copy(qrid_hbm.at[pl.ds(qbase, per_wq)], qidx_v)
        pltpu.async_copy(zq_hbm.at[qidx_v], qrows_v, qsem).wait()
        pltpu.sync_copy(qrows_v, outz.at[pl.ds(qbase, per_wq)])

    return gk


def _attn_call(e2, wg, zqg, rw, w1t, w2t, b2, u2, bb, m):
    b = zqg.shape[0]
    dim = e2.shape[-1]
    r = bb * m
    body = functools.partial(_attn_block_kernel, (bb, m))
    return pl.pallas_call(
        body,
        grid=(b // bb,),
        in_specs=[
            pl.BlockSpec((r, dim), lambda i: (i, 0)),
            pl.BlockSpec((r, dim // 2), lambda i: (i, 0)),
            pl.BlockSpec((bb, dim), lambda i: (i, 0)),
            pl.BlockSpec((r, 1), lambda i: (i, 0)),
            pl.BlockSpec(w1t.shape, lambda i: (0, 0)),
            pl.BlockSpec(w2t.shape, lambda i: (0, 0)),
            pl.BlockSpec(b2.shape, lambda i: (0, 0)),
            pl.BlockSpec(u2.shape, lambda i: (0, 0)),
        ],
        out_specs=pl.BlockSpec((bb, dim), lambda i: (i, 0)),
        out_shape=jax.ShapeDtypeStruct((b, dim), jnp.float32),
    )(e2, wg, zqg, rw, w1t, w2t, b2, u2)


def kernel(batch_nei_rid, batch_nei_e_emb, batch_nei_rw, batch_q_rid,
           w_r_table, zq_table, W_attn, b_attn, W_ua, b_ua):
    b, m, dim = batch_nei_e_emb.shape
    rid = batch_nei_rid.reshape(-1).astype(jnp.int32)
    qrid = batch_q_rid.astype(jnp.int32)
    nv = w_r_table.shape[0]
    pad = (-nv) % 8
    wr_pad = jnp.pad(w_r_table.astype(jnp.float32), ((0, pad), (0, 0)),
                     constant_values=1.0)
    nhat_packed = _pack_table(_normalize_call(wr_pad))
    e2 = batch_nei_e_emb.reshape(b * m, dim)
    w1t = W_attn[:, :dim].T
    w2t = W_attn[:, dim:].T.astype(jnp.bfloat16)
    b2 = b_attn.reshape(1, -1)
    u2 = W_ua.reshape(-1, 1).astype(jnp.bfloat16)
    rw2 = batch_nei_rw.reshape(b * m, 1)
    zq32 = zq_table.astype(jnp.float32)
    nc = _NCHUNK
    cb = b // nc
    cr = cb * m
    gather = _make_sc_gather(cr, cb, dim)
    gathered = [
        gather(lax.dynamic_slice_in_dim(rid, c * cr, cr, 0),
               lax.dynamic_slice_in_dim(qrid, c * cb, cb, 0),
               nhat_packed, zq32)
        for c in range(nc)
    ]
    outs = [
        _attn_call(lax.dynamic_slice_in_dim(e2, c * cr, cr, 0), wg_c, zqg_c,
                   lax.dynamic_slice_in_dim(rw2, c * cr, cr, 0),
                   w1t, w2t, b2, u2, _BB, m)
        for c, (wg_c, zqg_c) in enumerate(gathered)
    ]
    return jnp.concatenate(outs, axis=0)


# speedup vs baseline: 1.6114x; 1.6114x over previous
import functools

import jax
import jax.numpy as jnp
from jax import lax
from jax.experimental import pallas as pl
from jax.experimental.pallas import tpu as pltpu
from jax.experimental.pallas import tpu_sc as plsc

_BB = 128
_CH = 128
_NCHUNK = 1


def _attn_block_kernel(nm, e_ref, wg_ref, zqg_ref, rw_ref, w1t_ref, w2t_ref,
                       b_ref, u_ref, out_ref):
    bb, m = nm
    e = e_ref[...]
    nh = _unpack_rows(wg_ref[...])
    dim = e.shape[-1]
    ones_c = jnp.ones((dim, 1), jnp.float32)
    dew = jnp.dot(e * nh, ones_c, preferred_element_type=jnp.float32)
    etr = e - dew * nh
    x = jnp.dot(etr.astype(jnp.bfloat16), w2t_ref[...],
                preferred_element_type=jnp.float32)
    zw = jnp.dot(zqg_ref[...], w1t_ref[...],
                 preferred_element_type=jnp.float32) + b_ref[...]
    x2 = x + jnp.broadcast_to(zw[:, None, :], (bb, m, zw.shape[-1])
                              ).reshape(bb * m, zw.shape[-1])
    t2 = jnp.tanh(x2.astype(jnp.bfloat16))
    logits = jnp.dot(t2, u_ref[...], preferred_element_type=jnp.float32)
    l3 = logits.reshape(bb, m, 1)
    mx = jnp.max(l3, axis=1, keepdims=True)
    ex = jnp.exp(l3 - mx)
    alpha = ex / jnp.sum(ex, axis=1, keepdims=True)
    attn = alpha + rw_ref[...].reshape(bb, m, 1)
    etr3 = etr.reshape(bb, m, dim)
    out_ref[...] = jnp.sum(attn * etr3, axis=1)


def _normalize_kernel(w_ref, out_ref):
    w = w_ref[...]
    ssq = jnp.sum(w * w, axis=-1, keepdims=True)
    out_ref[...] = w / jnp.maximum(jnp.sqrt(ssq), 1e-12)


def _normalize_call(w):
    return pl.pallas_call(
        _normalize_kernel,
        out_shape=jax.ShapeDtypeStruct(w.shape, jnp.float32),
    )(w)


def _pack_table(nhat):
    h = nhat.shape[-1] // 2
    stored = jnp.stack([nhat[:, :h], nhat[:, h:]], axis=-1).astype(jnp.bfloat16)
    return jax.lax.bitcast_convert_type(stored, jnp.int32)


def _unpack_rows(u):
    f_lo = jax.lax.bitcast_convert_type(jnp.left_shift(u, 16), jnp.float32)
    f_hi = jax.lax.bitcast_convert_type(u & jnp.int32(-65536), jnp.float32)
    return jnp.concatenate([f_lo, f_hi], axis=-1)


@functools.cache
def _make_sc_gather(n_rows, n_q, dim):
    info = plsc.get_sparse_core_info()
    nc = info.num_cores
    nw = nc * info.num_subcores
    per_w = n_rows // nw
    n_ch = per_w // _CH
    per_wq = n_q // nw
    mesh = plsc.VectorSubcoreMesh(core_axis_name="c", subcore_axis_name="s")

    @functools.partial(
        pl.kernel, mesh=mesh,
        out_type=[jax.ShapeDtypeStruct((n_rows, dim // 2), jnp.int32),
                  jax.ShapeDtypeStruct((n_q, dim), jnp.float32)],
        scratch_types=[pltpu.VMEM((per_w,), jnp.int32),
                       pltpu.VMEM((_CH, dim // 2), jnp.int32),
                       pltpu.VMEM((_CH, dim // 2), jnp.int32),
                       pltpu.VMEM((_CH, dim // 2), jnp.int32),
                       pltpu.VMEM((per_wq,), jnp.int32),
                       pltpu.VMEM((per_wq, dim), jnp.float32),
                       pltpu.SemaphoreType.DMA,
                       pltpu.SemaphoreType.DMA,
                       pltpu.SemaphoreType.DMA,
                       pltpu.SemaphoreType.DMA,
                       pltpu.SemaphoreType.DMA,
                       pltpu.SemaphoreType.DMA,
                       pltpu.SemaphoreType.DMA],
    )
    def gk(rid_hbm, qrid_hbm, wr_hbm, zq_hbm, outw, outz,
           idx_all, rows0, rows1, rows2, qidx_v, qrows_v,
           g0, g1, g2, s0, s1, s2, qsem):
        wid = lax.axis_index("s") * nc + lax.axis_index("c")
        base0 = wid * per_w
        pltpu.sync_copy(rid_hbm.at[pl.ds(pl.multiple_of(base0, 8), per_w)],
                        idx_all)
        nbuf = 3
        bufs = (rows0, rows1, rows2)
        gsems = (g0, g1, g2)
        ssems = (s0, s1, s2)

        def gather_start(i, buf, sem):
            return pltpu.async_copy(
                wr_hbm.at[idx_all.at[pl.ds(i * _CH, _CH)]], buf, sem)

        def store_start(i, buf, sem):
            dst = pl.multiple_of(base0 + i * _CH, 8)
            return pltpu.async_copy(buf, outw.at[pl.ds(dst, _CH)], sem)

        gathers = [None] * nbuf
        stores = [None] * nbuf
        for i in range(min(nbuf - 1, n_ch)):
            gathers[i % nbuf] = gather_start(i, bufs[i % nbuf], gsems[i % nbuf])
        for i in range(n_ch):
            cur = i % nbuf
            gathers[cur].wait()
            j = i + nbuf - 1
            if j < n_ch:
                jb = j % nbuf
                if stores[jb] is not None:
                    stores[jb].wait()
                gathers[jb] = gather_start(j, bufs[jb], gsems[jb])
            stores[cur] = store_start(i, bufs[cur], ssems[cur])
        for st in stores:
            if st is not None:
                st.wait()

        qbase = pl.multiple_of(wid * per_wq, 8)
        pltpu.sync_copy(qrid_hbm.at[pl.ds(qbase, per_wq)], qidx_v)
        pltpu.async_copy(zq_hbm.at[qidx_v], qrows_v, qsem).wait()
        pltpu.sync_copy(qrows_v, outz.at[pl.ds(qbase, per_wq)])

    return gk


def _attn_call(e2, wg, zqg, rw, w1t, w2t, b2, u2, bb, m):
    b = zqg.shape[0]
    dim = e2.shape[-1]
    r = bb * m
    body = functools.partial(_attn_block_kernel, (bb, m))
    return pl.pallas_call(
        body,
        grid=(b // bb,),
        in_specs=[
            pl.BlockSpec((r, dim), lambda i: (i, 0)),
            pl.BlockSpec((r, dim // 2), lambda i: (i, 0)),
            pl.BlockSpec((bb, dim), lambda i: (i, 0)),
            pl.BlockSpec((r, 1), lambda i: (i, 0)),
            pl.BlockSpec(w1t.shape, lambda i: (0, 0)),
            pl.BlockSpec(w2t.shape, lambda i: (0, 0)),
            pl.BlockSpec(b2.shape, lambda i: (0, 0)),
            pl.BlockSpec(u2.shape, lambda i: (0, 0)),
        ],
        out_specs=pl.BlockSpec((bb, dim), lambda i: (i, 0)),
        out_shape=jax.ShapeDtypeStruct((b, dim), jnp.float32),
    )(e2, wg, zqg, rw, w1t, w2t, b2, u2)


def kernel(batch_nei_rid, batch_nei_e_emb, batch_nei_rw, batch_q_rid,
           w_r_table, zq_table, W_attn, b_attn, W_ua, b_ua):
    b, m, dim = batch_nei_e_emb.shape
    rid = batch_nei_rid.reshape(-1).astype(jnp.int32)
    qrid = batch_q_rid.astype(jnp.int32)
    nv = w_r_table.shape[0]
    pad = (-nv) % 8
    wr_pad = jnp.pad(w_r_table.astype(jnp.float32), ((0, pad), (0, 0)),
                     constant_values=1.0)
    nhat_packed = _pack_table(_normalize_call(wr_pad))
    e2 = batch_nei_e_emb.reshape(b * m, dim)
    w1t = W_attn[:, :dim].T
    w2t = W_attn[:, dim:].T.astype(jnp.bfloat16)
    b2 = b_attn.reshape(1, -1)
    u2 = W_ua.reshape(-1, 1).astype(jnp.bfloat16)
    rw2 = batch_nei_rw.reshape(b * m, 1)
    zq32 = zq_table.astype(jnp.float32)
    nc = _NCHUNK
    cb = b // nc
    cr = cb * m
    gather = _make_sc_gather(cr, cb, dim)
    gathered = [
        gather(lax.dynamic_slice_in_dim(rid, c * cr, cr, 0),
               lax.dynamic_slice_in_dim(qrid, c * cb, cb, 0),
               nhat_packed, zq32)
        for c in range(nc)
    ]
    outs = [
        _attn_call(lax.dynamic_slice_in_dim(e2, c * cr, cr, 0), wg_c, zqg_c,
                   lax.dynamic_slice_in_dim(rw2, c * cr, cr, 0),
                   w1t, w2t, b2, u2, _BB, m)
        for c, (wg_c, zqg_c) in enumerate(gathered)
    ]
    return jnp.concatenate(outs, axis=0)
